# Initial kernel scaffold; baseline (speedup 1.0000x reference)
import jax
import jax.numpy as jnp
from jax.experimental import pallas as pl


def kernel(inputs, table):
    raise NotImplementedError("write your pallas kernel here")


# speedup vs baseline: 1.2977x; 1.2977x over previous
import jax
import jax.numpy as jnp
from jax.experimental import pallas as pl
from jax.experimental.pallas import tpu as pltpu

_NUM_E = 1024
_DIM = 64
_CHUNK = 2048
_COMMIT = 0.25


def _body(x_ref, tt_ref, t1_ref, t2_ref, t3_ref, q_ref, i_ref, loss_ref, xt_s):
    x = x_ref[0]
    tt = tt_ref[...]
    t1 = t1_ref[...]
    t2 = t2_ref[...]
    t3 = t3_ref[...]

    xt_s[...] = x.T
    xt = xt_s[...]

    def dot(a, b):
        return jax.lax.dot_general(
            a, b, (((1,), (0,)), ((), ())),
            preferred_element_type=jnp.float32)

    mm = dot(xt, tt)

    e2 = jnp.sum(tt * tt, axis=0, keepdims=True)
    xtv = x.T
    x2 = jnp.sum(xtv * xtv, axis=1, keepdims=True)

    d2 = (x2 + e2) - 2.0 * mm
    m0 = jnp.maximum(d2, 0.0)
    dist = jnp.where(m0 == 0.0, 0.0, m0 * jax.lax.rsqrt(m0))

    mind = jnp.min(dist, axis=1, keepdims=True)
    iota1 = jax.lax.broadcasted_iota(jnp.int32, dist.shape, 1)
    idx = jnp.min(jnp.where(dist == mind, iota1, _NUM_E), axis=1)

    onehot = (iota1 == idx[:, None]).astype(jnp.float32)
    q = (dot(onehot, t1) + dot(onehot, t2)) + dot(onehot, t3)

    qst = xt + (q - xt)
    q_ref[0] = qst.T
    i_ref[0] = idx[:, None]

    diff = q - xt
    part = jnp.sum(diff * diff, keepdims=True)
    first = (pl.program_id(0) == 0) & (pl.program_id(1) == 0)

    @pl.when(first)
    def _init():
        loss_ref[...] = part

    @pl.when(jnp.logical_not(first))
    def _acc():
        loss_ref[...] += part


def kernel(inputs, table):
    B, C, L, H, W = inputs.shape
    N = L * H * W
    x3 = inputs.reshape(B, C, N)

    t1 = table.astype(jnp.bfloat16).astype(jnp.float32)
    r1 = table - t1
    t2 = r1.astype(jnp.bfloat16).astype(jnp.float32)
    t3 = (r1 - t2).astype(jnp.bfloat16).astype(jnp.float32)

    t_spec = pl.BlockSpec((_NUM_E, _DIM), lambda b, c: (0, 0))
    q3, idx3, loss_s = pl.pallas_call(
        _body,
        grid=(B, N // _CHUNK),
        in_specs=[
            pl.BlockSpec((1, C, _CHUNK), lambda b, c: (b, 0, c)),
            pl.BlockSpec((_DIM, _NUM_E), lambda b, c: (0, 0)),
            t_spec, t_spec, t_spec,
        ],
        out_specs=[
            pl.BlockSpec((1, C, _CHUNK), lambda b, c: (b, 0, c)),
            pl.BlockSpec((1, _CHUNK, 1), lambda b, c: (b, c, 0)),
            pl.BlockSpec((1, 1), lambda b, c: (0, 0)),
        ],
        out_shape=[
            jax.ShapeDtypeStruct((B, C, N), jnp.float32),
            jax.ShapeDtypeStruct((B, N, 1), jnp.int32),
            jax.ShapeDtypeStruct((1, 1), jnp.float32),
        ],
        scratch_shapes=[pltpu.VMEM((_CHUNK, _DIM), jnp.float32)],
    )(x3, table.T, t1, t2, t3)

    quantized_st = q3.reshape(B, C, L, H, W)
    encoding_indices = idx3.reshape(B * N, 1)
    mean_sq = loss_s[0, 0] / inputs.size
    loss = mean_sq + _COMMIT * mean_sq
    return (quantized_st, loss, encoding_indices)


# speedup vs baseline: 1.4387x; 1.1087x over previous
import jax
import jax.numpy as jnp
from jax.experimental import pallas as pl
from jax.experimental.pallas import tpu as pltpu

_NUM_E = 1024
_DIM = 64
_CHUNK = 2048
_COMMIT = 0.25


def _body(x_ref, tt_ref, t1_ref, t2_ref, q_ref, i_ref, loss_ref, xt_s):
    x = x_ref[0]
    tt = tt_ref[...]
    t1 = t1_ref[...]
    t2 = t2_ref[...]

    xt_s[...] = x.T
    xt = xt_s[...]

    def dot(a, b):
        return jax.lax.dot_general(
            a, b, (((1,), (0,)), ((), ())),
            preferred_element_type=jnp.float32)

    mm = dot(xt, tt)

    e2 = jnp.sum(tt * tt, axis=0, keepdims=True)
    xtv = x.T
    x2 = jnp.sum(xtv * xtv, axis=1, keepdims=True)

    d2 = (x2 + e2) - 2.0 * mm
    m0 = jnp.maximum(d2, 0.0)
    dist = jnp.where(m0 == 0.0, 0.0, m0 * jax.lax.rsqrt(m0))

    mind = jnp.min(dist, axis=1, keepdims=True)
    iota1 = jax.lax.broadcasted_iota(jnp.int32, dist.shape, 1)
    idx = jnp.min(jnp.where(dist == mind, iota1, _NUM_E), axis=1)

    onehot = (iota1 == idx[:, None]).astype(jnp.float32)
    q = dot(onehot, t1) + dot(onehot, t2)

    qst = xt + (q - xt)
    q_ref[0] = qst.T
    i_ref[0] = idx[:, None]

    diff = q - xt
    part = jnp.sum(diff * diff, keepdims=True)
    first = (pl.program_id(0) == 0) & (pl.program_id(1) == 0)

    @pl.when(first)
    def _init():
        loss_ref[...] = part

    @pl.when(jnp.logical_not(first))
    def _acc():
        loss_ref[...] += part


def kernel(inputs, table):
    B, C, L, H, W = inputs.shape
    N = L * H * W
    x3 = inputs.reshape(B, C, N)

    t1 = table.astype(jnp.bfloat16).astype(jnp.float32)
    t2 = (table - t1).astype(jnp.bfloat16).astype(jnp.float32)

    t_spec = pl.BlockSpec((_NUM_E, _DIM), lambda b, c: (0, 0))
    q3, idx3, loss_s = pl.pallas_call(
        _body,
        grid=(B, N // _CHUNK),
        in_specs=[
            pl.BlockSpec((1, C, _CHUNK), lambda b, c: (b, 0, c)),
            pl.BlockSpec((_DIM, _NUM_E), lambda b, c: (0, 0)),
            t_spec, t_spec,
        ],
        out_specs=[
            pl.BlockSpec((1, C, _CHUNK), lambda b, c: (b, 0, c)),
            pl.BlockSpec((1, _CHUNK, 1), lambda b, c: (b, c, 0)),
            pl.BlockSpec((1, 1), lambda b, c: (0, 0)),
        ],
        out_shape=[
            jax.ShapeDtypeStruct((B, C, N), jnp.float32),
            jax.ShapeDtypeStruct((B, N, 1), jnp.int32),
            jax.ShapeDtypeStruct((1, 1), jnp.float32),
        ],
        scratch_shapes=[pltpu.VMEM((_CHUNK, _DIM), jnp.float32)],
    )(x3, table.T, t1, t2)

    quantized_st = q3.reshape(B, C, L, H, W)
    encoding_indices = idx3.reshape(B * N, 1)
    mean_sq = loss_s[0, 0] / inputs.size
    loss = mean_sq + _COMMIT * mean_sq
    return (quantized_st, loss, encoding_indices)
